# Initial kernel scaffold; baseline (speedup 1.0000x reference)
import jax
import jax.numpy as jnp
from jax.experimental import pallas as pl


def kernel(inputs, clean_gate, noise_gate, expert_W, patch_h, patch_w):
    raise NotImplementedError("write your pallas kernel here")


# speedup vs baseline: 3.3805x; 3.3805x over previous
import jax
import jax.numpy as jnp
from jax.experimental import pallas as pl
from jax.experimental.pallas import tpu as pltpu

NUM_EXPERTS = 8
TOP_K = 2
NEG = -1e30


def _round_f16(x):
    r = jax.lax.bitcast_convert_type(x, jnp.int32)
    r = (r + 0x0FFF + ((r >> 13) & 1)) & ~0x1FFF
    return jax.lax.bitcast_convert_type(r, jnp.float32)


def _moe_block(flat_ref, gate_ref, w_ref, res_ref, idx_ref, gates_ref, load_ref):
    i = pl.program_id(0)
    x = flat_ref[...]
    g = gate_ref[...]
    logits = jnp.dot(x, g, preferred_element_type=jnp.float32)
    col = jax.lax.broadcasted_iota(jnp.int32, logits.shape, 1)
    valid = col < NUM_EXPERTS
    logits = jnp.where(valid, logits, NEG)

    v1 = jnp.max(logits, axis=1, keepdims=True)
    i1 = jnp.min(jnp.where(logits == v1, col, 128), axis=1, keepdims=True)
    l2 = jnp.where(col == i1, NEG, logits)
    v2 = jnp.max(l2, axis=1, keepdims=True)
    i2 = jnp.min(jnp.where(l2 == v2, col, 128), axis=1, keepdims=True)

    e2 = jnp.exp(v2 - v1)
    g1 = 1.0 / (1.0 + e2)
    g2 = e2 / (1.0 + e2)
    g1 = _round_f16(g1)
    g2 = _round_f16(g2)

    gates_blk = jnp.where(col == i1, g1, 0.0) + jnp.where(col == i2, g2, 0.0)
    gates_ref[...] = gates_blk
    idx_ref[...] = jnp.where(col == 0, i1, jnp.where(col == 1, i2, 0))

    load_part = jnp.sum((gates_blk > 0).astype(jnp.int32), axis=0, keepdims=True)

    @pl.when(i == 0)
    def _():
        load_ref[...] = jnp.zeros_like(load_ref)

    load_ref[...] += load_part

    acc = jnp.zeros((x.shape[0], w_ref.shape[2]), dtype=jnp.float32)
    for e in range(NUM_EXPERTS):
        ind = ((i1 == e) | (i2 == e)).astype(jnp.float32)
        acc = acc + ind * jnp.dot(x, w_ref[e], preferred_element_type=jnp.float32)
    res_ref[...] = acc


def kernel(inputs, clean_gate, noise_gate, expert_W, patch_h, patch_w):
    b, s, dim = inputs.shape
    flat = inputs.reshape(-1, dim)
    T = flat.shape[0]
    R = 1024
    nblk = T // R
    d_out = expert_W.shape[-1]

    gate_pad = jnp.zeros((dim, 128), jnp.float32).at[:, :NUM_EXPERTS].set(clean_gate)

    res, idx, gates, load = pl.pallas_call(
        _moe_block,
        grid=(nblk,),
        in_specs=[
            pl.BlockSpec((R, dim), lambda i: (i, 0)),
            pl.BlockSpec((dim, 128), lambda i: (0, 0)),
            pl.BlockSpec((NUM_EXPERTS, dim, d_out), lambda i: (0, 0, 0)),
        ],
        out_specs=[
            pl.BlockSpec((R, d_out), lambda i: (i, 0)),
            pl.BlockSpec((R, 128), lambda i: (i, 0)),
            pl.BlockSpec((R, 128), lambda i: (i, 0)),
            pl.BlockSpec((1, 128), lambda i: (0, 0)),
        ],
        out_shape=[
            jax.ShapeDtypeStruct((T, d_out), jnp.float32),
            jax.ShapeDtypeStruct((T, 128), jnp.int32),
            jax.ShapeDtypeStruct((T, 128), jnp.float32),
            jax.ShapeDtypeStruct((1, 128), jnp.int32),
        ],
    )(flat, gate_pad, expert_W)

    return (res.reshape(b, s, d_out), idx[:, :TOP_K], gates[:, :NUM_EXPERTS],
            load[0, :NUM_EXPERTS])
